# baseline (speedup 1.0000x reference)
import jax
import jax.numpy as jnp
from jax import lax
from jax.experimental import pallas as pl
from jax.experimental.pallas import tpu as pltpu
from jax.experimental.pallas import tpu_sc as plsc

NC = 2
NS = 16
NW = NC * NS
LANES = 16

N = 10000
D = 128
E = 320000
N_PAD = 10240
CHUNK = 128
ROWS_PER_TILE = N_PAD // NS
DEG_PER_TILE = N_PAD // NS
C = -(-E // (NW * CHUNK))
E_PAD = NW * C * CHUNK
SECC = 40
SEC = 2
C4 = SEC * SECC
E_PAD4 = NW * C4 * CHUNK

_mesh = lambda: plsc.VectorSubcoreMesh(core_axis_name="c", subcore_axis_name="s")


def _deg_body(dst_hbm, out_hbm, dst_v, ones_v, zero_v, deg_sh):
  c = lax.axis_index("c")
  s = lax.axis_index("s")
  w = c * NS + s
  pltpu.sync_copy(dst_hbm.at[w], dst_v)

  def fill(i, carry):
    ones_v[pl.ds(i * LANES, LANES)] = jnp.ones((LANES,), jnp.float32)
    return carry
  lax.fori_loop(0, CHUNK // LANES, fill, 0, unroll=8)

  def zfill(i, carry):
    zero_v[pl.ds(i * LANES, LANES)] = jnp.zeros((LANES,), jnp.float32)
    return carry
  lax.fori_loop(0, DEG_PER_TILE // LANES, zfill, 0, unroll=8)

  pltpu.sync_copy(zero_v, deg_sh.at[pl.ds(s * DEG_PER_TILE, DEG_PER_TILE)])
  plsc.subcore_barrier()

  def count(j, carry):
    pltpu.sync_copy(ones_v, deg_sh.at[dst_v.at[j]], add=True)
    return carry
  lax.fori_loop(0, C, count, 0)

  plsc.subcore_barrier()
  pltpu.sync_copy(deg_sh.at[pl.ds(s * DEG_PER_TILE, DEG_PER_TILE)],
                  out_hbm.at[c].at[pl.ds(s * DEG_PER_TILE, DEG_PER_TILE)])


def _deg_partials(dst3):
  k = pl.kernel(
      _deg_body,
      out_type=jax.ShapeDtypeStruct((NC, N_PAD), jnp.float32),
      mesh=_mesh(),
      scratch_types=[
          pltpu.VMEM((C, CHUNK), jnp.int32),
          pltpu.VMEM((CHUNK,), jnp.float32),
          pltpu.VMEM((DEG_PER_TILE,), jnp.float32),
          pltpu.VMEM_SHARED((N_PAD,), jnp.float32),
      ],
  )
  return k(dst3)


def _agg_body(g_hbm, src_hbm, dst_hbm, acc_out, src_v, dst_v, rows, acc_sh,
              sem0, sem1):
  c = lax.axis_index("c")
  s = lax.axis_index("s")
  w = c * NS + s

  def zero(i, carry):
    for k in range(D // LANES):
      rows[0, i, pl.ds(k * LANES, LANES)] = jnp.zeros((LANES,), jnp.float32)
    return carry
  lax.fori_loop(0, CHUNK, zero, 0, unroll=4)
  base = s * ROWS_PER_TILE
  for t in range(ROWS_PER_TILE // CHUNK):
    pltpu.sync_copy(rows.at[0], acc_sh.at[pl.ds(base + t * CHUNK, CHUNK)])
  plsc.subcore_barrier()

  sems = (sem0, sem1)

  for sec in range(SEC):
    pltpu.sync_copy(src_hbm.at[w].at[pl.ds(sec * SECC, SECC)], src_v)
    pltpu.sync_copy(dst_hbm.at[w].at[pl.ds(sec * SECC, SECC)], dst_v)

    pltpu.async_copy(g_hbm.at[src_v.at[0]], rows.at[0], sem0)

    def pair(p, carry):
      for b in range(2):
        j = p * 2 + b

        @pl.when(j + 1 < SECC)
        def _prefetch():
          nb = 1 - b
          pltpu.async_copy(g_hbm.at[src_v.at[j + 1]], rows.at[nb], sems[nb])

        pltpu.make_async_copy(g_hbm.at[src_v.at[j]], rows.at[b],
                              sems[b]).wait()
        pltpu.sync_copy(rows.at[b], acc_sh.at[dst_v.at[j]], add=True)
      return carry

    lax.fori_loop(0, SECC // 2, pair, 0)

  plsc.subcore_barrier()
  pltpu.sync_copy(acc_sh.at[pl.ds(base, ROWS_PER_TILE)],
                  acc_out.at[c].at[pl.ds(base, ROWS_PER_TILE)])


def _scatter_rows(g, src4, dst4):
  k = pl.kernel(
      _agg_body,
      out_type=jax.ShapeDtypeStruct((NC, N_PAD, D), jnp.float32),
      mesh=_mesh(),
      scratch_types=[
          pltpu.VMEM((SECC, CHUNK), jnp.int32),
          pltpu.VMEM((SECC, CHUNK), jnp.int32),
          pltpu.VMEM((2, CHUNK, D), jnp.float32),
          pltpu.VMEM_SHARED((N_PAD, D), jnp.float32),
          pltpu.SemaphoreType.DMA,
          pltpu.SemaphoreType.DMA,
      ],
  )
  return k(g, src4, dst4)


BLK = 512
GRID = N_PAD // BLK


def _dinv_col(dp):
  deg = jnp.sum(dp, axis=1, keepdims=True) + 1.0
  return lax.rsqrt(deg)


def _mm1_body(x_ref, w_ref, dp_ref, h_ref, g_ref):
  h = jnp.dot(x_ref[...], w_ref[...], preferred_element_type=jnp.float32)
  dinv = _dinv_col(dp_ref[...])
  h_ref[...] = h
  g_ref[...] = h * dinv


def _mid_body(acc_ref, h_ref, dp_ref, b_ref, w_ref, h2_ref, g2_ref):
  a = acc_ref[0] + acc_ref[1]
  dinv = _dinv_col(dp_ref[...])
  z = jnp.maximum(a * dinv + h_ref[...] * (dinv * dinv) + b_ref[...], 0.0)
  h2 = jnp.dot(z, w_ref[...], preferred_element_type=jnp.float32)
  h2_ref[...] = h2
  g2_ref[...] = h2 * dinv


def _fin_body(acc_ref, h_ref, dp_ref, b_ref, o_ref):
  a = acc_ref[0] + acc_ref[1]
  dinv = _dinv_col(dp_ref[...])
  o_ref[...] = a * dinv + h_ref[...] * (dinv * dinv) + b_ref[...]


def _mm1(x_pad, W1, dpT):
  return pl.pallas_call(
      _mm1_body,
      grid=(GRID,),
      in_specs=[
          pl.BlockSpec((BLK, D), lambda i: (i, 0)),
          pl.BlockSpec((D, D), lambda i: (0, 0)),
          pl.BlockSpec((BLK, NC), lambda i: (i, 0)),
      ],
      out_specs=[
          pl.BlockSpec((BLK, D), lambda i: (i, 0)),
          pl.BlockSpec((BLK, D), lambda i: (i, 0)),
      ],
      out_shape=[
          jax.ShapeDtypeStruct((N_PAD, D), jnp.float32),
          jax.ShapeDtypeStruct((N_PAD, D), jnp.float32),
      ],
  )(x_pad, W1, dpT)


def _mid(acc, h1, dpT, b1, W2):
  return pl.pallas_call(
      _mid_body,
      grid=(GRID,),
      in_specs=[
          pl.BlockSpec((NC, BLK, D), lambda i: (0, i, 0)),
          pl.BlockSpec((BLK, D), lambda i: (i, 0)),
          pl.BlockSpec((BLK, NC), lambda i: (i, 0)),
          pl.BlockSpec((1, D), lambda i: (0, 0)),
          pl.BlockSpec((D, D), lambda i: (0, 0)),
      ],
      out_specs=[
          pl.BlockSpec((BLK, D), lambda i: (i, 0)),
          pl.BlockSpec((BLK, D), lambda i: (i, 0)),
      ],
      out_shape=[
          jax.ShapeDtypeStruct((N_PAD, D), jnp.float32),
          jax.ShapeDtypeStruct((N_PAD, D), jnp.float32),
      ],
  )(acc, h1, dpT, b1.reshape(1, D), W2)


def _fin(acc, h2, dpT, b2):
  return pl.pallas_call(
      _fin_body,
      grid=(GRID,),
      in_specs=[
          pl.BlockSpec((NC, BLK, D), lambda i: (0, i, 0)),
          pl.BlockSpec((BLK, D), lambda i: (i, 0)),
          pl.BlockSpec((BLK, NC), lambda i: (i, 0)),
          pl.BlockSpec((1, D), lambda i: (0, 0)),
      ],
      out_specs=pl.BlockSpec((BLK, D), lambda i: (i, 0)),
      out_shape=jax.ShapeDtypeStruct((N_PAD, D), jnp.float32),
  )(acc, h2, dpT, b2.reshape(1, D))


@jax.jit
def kernel(x, edge_index, W1, b1, W2, b2):
  src = edge_index[0]
  dst = edge_index[1]
  pad3 = E_PAD - E
  dst3 = jnp.concatenate(
      [dst, jnp.full((pad3,), N, jnp.int32)]).reshape(NW, C, CHUNK)
  pad4 = E_PAD4 - E
  src4 = jnp.concatenate(
      [src, jnp.full((pad4,), N, jnp.int32)]).reshape(NW, C4, CHUNK)
  dst4 = jnp.concatenate(
      [dst, jnp.full((pad4,), N, jnp.int32)]).reshape(NW, C4, CHUNK)

  x_pad = jnp.zeros((N_PAD, D), jnp.float32).at[:N].set(x)

  dp = _deg_partials(dst3)
  dpT = dp.T

  h1, g1 = _mm1(x_pad, W1, dpT)
  acc1 = _scatter_rows(g1, src4, dst4)
  h2, g2 = _mid(acc1, h1, dpT, b1, W2)
  acc2 = _scatter_rows(g2, src4, dst4)
  out = _fin(acc2, h2, dpT, b2)
  return out[:N]


# speedup vs baseline: 3.4619x; 3.4619x over previous
import jax
import jax.numpy as jnp
from jax import lax
from jax.experimental import pallas as pl
from jax.experimental.pallas import tpu as pltpu
from jax.experimental.pallas import tpu_sc as plsc

NC = 2
NS = 16
NW = NC * NS
LANES = 16

N = 10000
D = 128
E = 320000
N_PAD = 10240
CHUNK = 128
ROWS_PER_TILE = N_PAD // NS
DEG_PER_TILE = N_PAD // NS
C = -(-E // (NW * CHUNK))
E_PAD = NW * C * CHUNK
SECC = 40
SEC = 2
C4 = SEC * SECC
E_PAD4 = NW * C4 * CHUNK

_mesh = lambda: plsc.VectorSubcoreMesh(core_axis_name="c", subcore_axis_name="s")


def _deg_body(dst_hbm, out_hbm, dst_v, ones_v, zero_v, deg_sh):
  c = lax.axis_index("c")
  s = lax.axis_index("s")
  w = c * NS + s
  pltpu.sync_copy(dst_hbm.at[w], dst_v)

  def fill(i, carry):
    ones_v[pl.ds(i * LANES, LANES)] = jnp.ones((LANES,), jnp.float32)
    return carry
  lax.fori_loop(0, CHUNK // LANES, fill, 0, unroll=8)

  def zfill(i, carry):
    zero_v[pl.ds(i * LANES, LANES)] = jnp.zeros((LANES,), jnp.float32)
    return carry
  lax.fori_loop(0, DEG_PER_TILE // LANES, zfill, 0, unroll=8)

  pltpu.sync_copy(zero_v, deg_sh.at[pl.ds(s * DEG_PER_TILE, DEG_PER_TILE)])
  plsc.subcore_barrier()

  def count(j, carry):
    pltpu.sync_copy(ones_v, deg_sh.at[dst_v.at[j]], add=True)
    return carry
  lax.fori_loop(0, C, count, 0)

  plsc.subcore_barrier()
  pltpu.sync_copy(deg_sh.at[pl.ds(s * DEG_PER_TILE, DEG_PER_TILE)],
                  out_hbm.at[c].at[pl.ds(s * DEG_PER_TILE, DEG_PER_TILE)])


def _deg_partials(dst3):
  k = pl.kernel(
      _deg_body,
      out_type=jax.ShapeDtypeStruct((NC, N_PAD), jnp.float32),
      mesh=_mesh(),
      scratch_types=[
          pltpu.VMEM((C, CHUNK), jnp.int32),
          pltpu.VMEM((CHUNK,), jnp.float32),
          pltpu.VMEM((DEG_PER_TILE,), jnp.float32),
          pltpu.VMEM_SHARED((N_PAD,), jnp.float32),
      ],
  )
  return k(dst3)


def _agg_body(g_hbm, src_hbm, dst_hbm, acc_out, src_v, dst_v, rows, acc_sh,
              sem0, sem1):
  c = lax.axis_index("c")
  s = lax.axis_index("s")
  w = c * NS + s

  def zero(i, carry):
    for k in range(D // LANES):
      rows[0, i, pl.ds(k * LANES, LANES)] = jnp.zeros((LANES,), jnp.float32)
    return carry
  lax.fori_loop(0, CHUNK, zero, 0, unroll=4)
  base = s * ROWS_PER_TILE
  for t in range(ROWS_PER_TILE // CHUNK):
    pltpu.sync_copy(rows.at[0], acc_sh.at[pl.ds(base + t * CHUNK, CHUNK)])
  plsc.subcore_barrier()

  sems = (sem0, sem1)

  for sec in range(SEC):
    pltpu.sync_copy(src_hbm.at[w].at[pl.ds(sec * SECC, SECC)], src_v)
    pltpu.sync_copy(dst_hbm.at[w].at[pl.ds(sec * SECC, SECC)], dst_v)

    pltpu.async_copy(g_hbm.at[src_v.at[0]], rows.at[0], sem0)

    def pair(p, carry):
      for b in range(2):
        j = p * 2 + b

        @pl.when(j + 1 < SECC)
        def _prefetch():
          nb = 1 - b
          pltpu.async_copy(g_hbm.at[src_v.at[j + 1]], rows.at[nb], sems[nb])

        pltpu.make_async_copy(g_hbm.at[src_v.at[j]], rows.at[b],
                              sems[b]).wait()
        pltpu.sync_copy(rows.at[b], acc_sh.at[dst_v.at[j]], add=True)
      return carry

    lax.fori_loop(0, SECC // 2, pair, 0)

  plsc.subcore_barrier()
  pltpu.sync_copy(acc_sh.at[pl.ds(base, ROWS_PER_TILE)],
                  acc_out.at[c].at[pl.ds(base, ROWS_PER_TILE)])


def _scatter_rows(g, src4, dst4):
  k = pl.kernel(
      _agg_body,
      out_type=jax.ShapeDtypeStruct((NC, N_PAD, D), jnp.float32),
      mesh=_mesh(),
      scratch_types=[
          pltpu.VMEM((SECC, CHUNK), jnp.int32),
          pltpu.VMEM((SECC, CHUNK), jnp.int32),
          pltpu.VMEM((2, CHUNK, D), jnp.float32),
          pltpu.VMEM_SHARED((N_PAD, D), jnp.float32),
          pltpu.SemaphoreType.DMA,
          pltpu.SemaphoreType.DMA,
      ],
  )
  return k(g, src4, dst4)


BLK = 512
GRID = N_PAD // BLK


def _dinv_col(dp):
  deg = jnp.sum(dp, axis=1, keepdims=True) + 1.0
  return lax.rsqrt(deg)


def _mm1_body(x_ref, w_ref, dp_ref, h_ref, g_ref):
  h = jnp.dot(x_ref[...], w_ref[...], preferred_element_type=jnp.float32)
  dinv = _dinv_col(dp_ref[...])
  h_ref[...] = h
  g_ref[...] = h * dinv


def _mid_body(acc_ref, h_ref, dp_ref, b_ref, w_ref, h2_ref, g2_ref):
  a = acc_ref[0] + acc_ref[1]
  dinv = _dinv_col(dp_ref[...])
  z = jnp.maximum(a * dinv + h_ref[...] * (dinv * dinv) + b_ref[...], 0.0)
  h2 = jnp.dot(z, w_ref[...], preferred_element_type=jnp.float32)
  h2_ref[...] = h2
  g2_ref[...] = h2 * dinv


def _fin_body(acc_ref, h_ref, dp_ref, b_ref, o_ref):
  a = acc_ref[0] + acc_ref[1]
  dinv = _dinv_col(dp_ref[...])
  o_ref[...] = a * dinv + h_ref[...] * (dinv * dinv) + b_ref[...]


def _mm1(x_pad, W1, dpT):
  return pl.pallas_call(
      _mm1_body,
      grid=(GRID,),
      in_specs=[
          pl.BlockSpec((BLK, D), lambda i: (i, 0)),
          pl.BlockSpec((D, D), lambda i: (0, 0)),
          pl.BlockSpec((BLK, NC), lambda i: (i, 0)),
      ],
      out_specs=[
          pl.BlockSpec((BLK, D), lambda i: (i, 0)),
          pl.BlockSpec((BLK, D), lambda i: (i, 0)),
      ],
      out_shape=[
          jax.ShapeDtypeStruct((N_PAD, D), jnp.float32),
          jax.ShapeDtypeStruct((N_PAD, D), jnp.float32),
      ],
  )(x_pad, W1, dpT)


def _mid(acc, h1, dpT, b1, W2):
  return pl.pallas_call(
      _mid_body,
      grid=(GRID,),
      in_specs=[
          pl.BlockSpec((NC, BLK, D), lambda i: (0, i, 0)),
          pl.BlockSpec((BLK, D), lambda i: (i, 0)),
          pl.BlockSpec((BLK, NC), lambda i: (i, 0)),
          pl.BlockSpec((1, D), lambda i: (0, 0)),
          pl.BlockSpec((D, D), lambda i: (0, 0)),
      ],
      out_specs=[
          pl.BlockSpec((BLK, D), lambda i: (i, 0)),
          pl.BlockSpec((BLK, D), lambda i: (i, 0)),
      ],
      out_shape=[
          jax.ShapeDtypeStruct((N_PAD, D), jnp.float32),
          jax.ShapeDtypeStruct((N_PAD, D), jnp.float32),
      ],
  )(acc, h1, dpT, b1.reshape(1, D), W2)


def _fin(acc, h2, dpT, b2):
  return pl.pallas_call(
      _fin_body,
      grid=(GRID,),
      in_specs=[
          pl.BlockSpec((NC, BLK, D), lambda i: (0, i, 0)),
          pl.BlockSpec((BLK, D), lambda i: (i, 0)),
          pl.BlockSpec((BLK, NC), lambda i: (i, 0)),
          pl.BlockSpec((1, D), lambda i: (0, 0)),
      ],
      out_specs=pl.BlockSpec((BLK, D), lambda i: (i, 0)),
      out_shape=jax.ShapeDtypeStruct((N_PAD, D), jnp.float32),
  )(acc, h2, dpT, b2.reshape(1, D))


@jax.jit
def kernel(x, edge_index, W1, b1, W2, b2):
  src = edge_index[0]
  dst = edge_index[1]
  spare = N_PAD - N
  pad3 = E_PAD - E
  fill3 = N + (jnp.arange(pad3, dtype=jnp.int32) % spare)
  dst3 = jnp.concatenate([dst, fill3]).reshape(NW, C, CHUNK)
  pad4 = E_PAD4 - E
  fill4 = N + (jnp.arange(pad4, dtype=jnp.int32) % spare)
  src4 = jnp.concatenate([src, fill4]).reshape(NW, C4, CHUNK)
  dst4 = jnp.concatenate([dst, fill4]).reshape(NW, C4, CHUNK)

  x_pad = jnp.zeros((N_PAD, D), jnp.float32).at[:N].set(x)

  dp = _deg_partials(dst3)
  dpT = dp.T

  h1, g1 = _mm1(x_pad, W1, dpT)
  acc1 = _scatter_rows(g1, src4, dst4)
  h2, g2 = _mid(acc1, h1, dpT, b1, W2)
  acc2 = _scatter_rows(g2, src4, dst4)
  out = _fin(acc2, h2, dpT, b2)
  return out[:N]


# speedup vs baseline: 3.6367x; 1.0505x over previous
import jax
import jax.numpy as jnp
from jax import lax
from jax.experimental import pallas as pl
from jax.experimental.pallas import tpu as pltpu
from jax.experimental.pallas import tpu_sc as plsc

NC = 2
NS = 16
NW = NC * NS
LANES = 16

N = 10000
D = 128
E = 320000
N_PAD = 10240
CHUNK = 128
ROWS_PER_TILE = N_PAD // NS
DEG_PER_TILE = N_PAD // NS
C = -(-E // (NW * CHUNK))
E_PAD = NW * C * CHUNK
SECC = 40
SEC = 2
C4 = SEC * SECC
E_PAD4 = NW * C4 * CHUNK

_mesh = lambda: plsc.VectorSubcoreMesh(core_axis_name="c", subcore_axis_name="s")


def _deg_body(dst_hbm, out_hbm, dst_v, ones_v, zero_v, deg_sh):
  c = lax.axis_index("c")
  s = lax.axis_index("s")
  w = c * NS + s
  pltpu.sync_copy(dst_hbm.at[w], dst_v)

  def fill(i, carry):
    ones_v[pl.ds(i * LANES, LANES)] = jnp.ones((LANES,), jnp.float32)
    return carry
  lax.fori_loop(0, CHUNK // LANES, fill, 0, unroll=8)

  def zfill(i, carry):
    zero_v[pl.ds(i * LANES, LANES)] = jnp.zeros((LANES,), jnp.float32)
    return carry
  lax.fori_loop(0, DEG_PER_TILE // LANES, zfill, 0, unroll=8)

  pltpu.sync_copy(zero_v, deg_sh.at[pl.ds(s * DEG_PER_TILE, DEG_PER_TILE)])
  plsc.subcore_barrier()

  def count(j, carry):
    pltpu.sync_copy(ones_v, deg_sh.at[dst_v.at[j]], add=True)
    return carry
  lax.fori_loop(0, C, count, 0)

  plsc.subcore_barrier()
  pltpu.sync_copy(deg_sh.at[pl.ds(s * DEG_PER_TILE, DEG_PER_TILE)],
                  out_hbm.at[c].at[pl.ds(s * DEG_PER_TILE, DEG_PER_TILE)])


def _deg_partials(dst3):
  k = pl.kernel(
      _deg_body,
      out_type=jax.ShapeDtypeStruct((NC, N_PAD), jnp.float32),
      mesh=_mesh(),
      scratch_types=[
          pltpu.VMEM((C, CHUNK), jnp.int32),
          pltpu.VMEM((CHUNK,), jnp.float32),
          pltpu.VMEM((DEG_PER_TILE,), jnp.float32),
          pltpu.VMEM_SHARED((N_PAD,), jnp.float32),
      ],
  )
  return k(dst3)


def _agg_body(g_hbm, src_hbm, dst_hbm, acc_out, src_v, dst_v, rows, acc_sh,
              sem0, sem1, sem2, sem3):
  c = lax.axis_index("c")
  s = lax.axis_index("s")
  w = c * NS + s

  def zero(i, carry):
    for k in range(D // LANES):
      rows[0, i, pl.ds(k * LANES, LANES)] = jnp.zeros((LANES,), jnp.float32)
    return carry
  lax.fori_loop(0, CHUNK, zero, 0, unroll=4)
  base = s * ROWS_PER_TILE
  for t in range(ROWS_PER_TILE // CHUNK):
    pltpu.sync_copy(rows.at[0], acc_sh.at[pl.ds(base + t * CHUNK, CHUNK)])
  plsc.subcore_barrier()

  gsems = (sem0, sem1)
  ssems = (sem2, sem3)

  def gather(j, b):
    pltpu.async_copy(g_hbm.at[src_v.at[j]], rows.at[b], gsems[b])

  def wait_gather(j, b):
    pltpu.make_async_copy(g_hbm.at[src_v.at[j]], rows.at[b], gsems[b]).wait()

  def scatter(j, b):
    pltpu.async_copy(rows.at[b], acc_sh.at[dst_v.at[j]], ssems[b], add=True)

  def wait_scatter(j, b):
    pltpu.make_async_copy(rows.at[b], acc_sh.at[dst_v.at[j]],
                          ssems[b]).wait()

  for sec in range(SEC):
    pltpu.sync_copy(src_hbm.at[w].at[pl.ds(sec * SECC, SECC)], src_v)
    pltpu.sync_copy(dst_hbm.at[w].at[pl.ds(sec * SECC, SECC)], dst_v)

    gather(0, 0)

    def pair(p, carry):
      for b in range(2):
        j = p * 2 + b
        nb = 1 - b

        @pl.when(j + 1 < SECC)
        def _prefetch():
          @pl.when(j >= 1)
          def _():
            wait_scatter(j - 1, nb)
          gather(j + 1, nb)

        wait_gather(j, b)
        scatter(j, b)
      return carry

    lax.fori_loop(0, SECC // 2, pair, 0)

    wait_scatter(SECC - 2, 0)
    wait_scatter(SECC - 1, 1)

  plsc.subcore_barrier()
  pltpu.sync_copy(acc_sh.at[pl.ds(base, ROWS_PER_TILE)],
                  acc_out.at[c].at[pl.ds(base, ROWS_PER_TILE)])


def _scatter_rows(g, src4, dst4):
  k = pl.kernel(
      _agg_body,
      out_type=jax.ShapeDtypeStruct((NC, N_PAD, D), jnp.float32),
      mesh=_mesh(),
      scratch_types=[
          pltpu.VMEM((SECC, CHUNK), jnp.int32),
          pltpu.VMEM((SECC, CHUNK), jnp.int32),
          pltpu.VMEM((2, CHUNK, D), jnp.float32),
          pltpu.VMEM_SHARED((N_PAD, D), jnp.float32),
          pltpu.SemaphoreType.DMA,
          pltpu.SemaphoreType.DMA,
          pltpu.SemaphoreType.DMA,
          pltpu.SemaphoreType.DMA,
      ],
  )
  return k(g, src4, dst4)


BLK = 1024
GRID = N_PAD // BLK


def _dinv_col(dp):
  deg = jnp.sum(dp, axis=1, keepdims=True) + 1.0
  return lax.rsqrt(deg)


def _mm1_body(x_ref, w_ref, dp_ref, h_ref, g_ref):
  h = jnp.dot(x_ref[...], w_ref[...], preferred_element_type=jnp.float32)
  dinv = _dinv_col(dp_ref[...])
  h_ref[...] = h
  g_ref[...] = h * dinv


def _mid_body(acc_ref, h_ref, dp_ref, b_ref, w_ref, h2_ref, g2_ref):
  a = acc_ref[0] + acc_ref[1]
  dinv = _dinv_col(dp_ref[...])
  z = jnp.maximum(a * dinv + h_ref[...] * (dinv * dinv) + b_ref[...], 0.0)
  h2 = jnp.dot(z, w_ref[...], preferred_element_type=jnp.float32)
  h2_ref[...] = h2
  g2_ref[...] = h2 * dinv


def _fin_body(acc_ref, h_ref, dp_ref, b_ref, o_ref):
  a = acc_ref[0] + acc_ref[1]
  dinv = _dinv_col(dp_ref[...])
  o_ref[...] = a * dinv + h_ref[...] * (dinv * dinv) + b_ref[...]


def _mm1(x_pad, W1, dpT):
  return pl.pallas_call(
      _mm1_body,
      grid=(GRID,),
      in_specs=[
          pl.BlockSpec((BLK, D), lambda i: (i, 0)),
          pl.BlockSpec((D, D), lambda i: (0, 0)),
          pl.BlockSpec((BLK, NC), lambda i: (i, 0)),
      ],
      out_specs=[
          pl.BlockSpec((BLK, D), lambda i: (i, 0)),
          pl.BlockSpec((BLK, D), lambda i: (i, 0)),
      ],
      out_shape=[
          jax.ShapeDtypeStruct((N_PAD, D), jnp.float32),
          jax.ShapeDtypeStruct((N_PAD, D), jnp.float32),
      ],
  )(x_pad, W1, dpT)


def _mid(acc, h1, dpT, b1, W2):
  return pl.pallas_call(
      _mid_body,
      grid=(GRID,),
      in_specs=[
          pl.BlockSpec((NC, BLK, D), lambda i: (0, i, 0)),
          pl.BlockSpec((BLK, D), lambda i: (i, 0)),
          pl.BlockSpec((BLK, NC), lambda i: (i, 0)),
          pl.BlockSpec((1, D), lambda i: (0, 0)),
          pl.BlockSpec((D, D), lambda i: (0, 0)),
      ],
      out_specs=[
          pl.BlockSpec((BLK, D), lambda i: (i, 0)),
          pl.BlockSpec((BLK, D), lambda i: (i, 0)),
      ],
      out_shape=[
          jax.ShapeDtypeStruct((N_PAD, D), jnp.float32),
          jax.ShapeDtypeStruct((N_PAD, D), jnp.float32),
      ],
  )(acc, h1, dpT, b1.reshape(1, D), W2)


def _fin(acc, h2, dpT, b2):
  return pl.pallas_call(
      _fin_body,
      grid=(GRID,),
      in_specs=[
          pl.BlockSpec((NC, BLK, D), lambda i: (0, i, 0)),
          pl.BlockSpec((BLK, D), lambda i: (i, 0)),
          pl.BlockSpec((BLK, NC), lambda i: (i, 0)),
          pl.BlockSpec((1, D), lambda i: (0, 0)),
      ],
      out_specs=pl.BlockSpec((BLK, D), lambda i: (i, 0)),
      out_shape=jax.ShapeDtypeStruct((N_PAD, D), jnp.float32),
  )(acc, h2, dpT, b2.reshape(1, D))


@jax.jit
def kernel(x, edge_index, W1, b1, W2, b2):
  src = edge_index[0]
  dst = edge_index[1]
  spare = N_PAD - N
  pad3 = E_PAD - E
  fill3 = N + (jnp.arange(pad3, dtype=jnp.int32) % spare)
  dst3 = jnp.concatenate([dst, fill3]).reshape(NW, C, CHUNK)
  pad4 = E_PAD4 - E
  fill4 = N + (jnp.arange(pad4, dtype=jnp.int32) % spare)
  src4 = jnp.concatenate([src, fill4]).reshape(NW, C4, CHUNK)
  dst4 = jnp.concatenate([dst, fill4]).reshape(NW, C4, CHUNK)

  dp = _deg_partials(dst3)
  dpT = dp.T

  h1, g1 = _mm1(x, W1, dpT)
  acc1 = _scatter_rows(g1, src4, dst4)
  h2, g2 = _mid(acc1, h1, dpT, b1, W2)
  acc2 = _scatter_rows(g2, src4, dst4)
  out = _fin(acc2, h2, dpT, b2)
  return out[:N]


# speedup vs baseline: 3.6989x; 1.0171x over previous
import jax
import jax.numpy as jnp
from jax import lax
from jax.experimental import pallas as pl
from jax.experimental.pallas import tpu as pltpu
from jax.experimental.pallas import tpu_sc as plsc

NC = 2
NS = 16
NW = NC * NS
LANES = 16

N = 10000
D = 128
E = 320000
N_PAD = 10240
CHUNK = 128
ROWS_PER_TILE = N_PAD // NS
DEG_PER_TILE = N_PAD // NS
C = -(-E // (NW * CHUNK))
E_PAD = NW * C * CHUNK
SECC = 40
SEC = 2
C4 = SEC * SECC
E_PAD4 = NW * C4 * CHUNK

_mesh = lambda: plsc.VectorSubcoreMesh(core_axis_name="c", subcore_axis_name="s")


def _deg_body(dst_hbm, out_hbm, dst_v, ones_v, zero_v, deg_sh):
  c = lax.axis_index("c")
  s = lax.axis_index("s")
  w = c * NS + s
  pltpu.sync_copy(dst_hbm.at[w], dst_v)

  def fill(i, carry):
    ones_v[pl.ds(i * LANES, LANES)] = jnp.ones((LANES,), jnp.float32)
    return carry
  lax.fori_loop(0, CHUNK // LANES, fill, 0, unroll=8)

  def zfill(i, carry):
    zero_v[pl.ds(i * LANES, LANES)] = jnp.zeros((LANES,), jnp.float32)
    return carry
  lax.fori_loop(0, DEG_PER_TILE // LANES, zfill, 0, unroll=8)

  pltpu.sync_copy(zero_v, deg_sh.at[pl.ds(s * DEG_PER_TILE, DEG_PER_TILE)])
  plsc.subcore_barrier()

  def count(j, carry):
    pltpu.sync_copy(ones_v, deg_sh.at[dst_v.at[j]], add=True)
    return carry
  lax.fori_loop(0, C, count, 0)

  plsc.subcore_barrier()
  pltpu.sync_copy(deg_sh.at[pl.ds(s * DEG_PER_TILE, DEG_PER_TILE)],
                  out_hbm.at[c].at[pl.ds(s * DEG_PER_TILE, DEG_PER_TILE)])


def _deg_partials(dst3):
  k = pl.kernel(
      _deg_body,
      out_type=jax.ShapeDtypeStruct((NC, N_PAD), jnp.float32),
      mesh=_mesh(),
      scratch_types=[
          pltpu.VMEM((C, CHUNK), jnp.int32),
          pltpu.VMEM((CHUNK,), jnp.float32),
          pltpu.VMEM((DEG_PER_TILE,), jnp.float32),
          pltpu.VMEM_SHARED((N_PAD,), jnp.float32),
      ],
  )
  return k(dst3)


def _agg_body(g_hbm, src_hbm, dst_hbm, acc_out, src_v, dst_v, rows, acc_sh,
              sem0, sem1, sem2, sem3):
  c = lax.axis_index("c")
  s = lax.axis_index("s")
  w = c * NS + s

  def zero(i, carry):
    for k in range(D // LANES):
      rows[0, i, pl.ds(k * LANES, LANES)] = jnp.zeros((LANES,), jnp.float32)
    return carry
  lax.fori_loop(0, CHUNK, zero, 0, unroll=4)
  base = s * ROWS_PER_TILE
  for t in range(ROWS_PER_TILE // CHUNK):
    pltpu.sync_copy(rows.at[0], acc_sh.at[pl.ds(base + t * CHUNK, CHUNK)])
  plsc.subcore_barrier()

  gsems = (sem0, sem1)
  ssems = (sem2, sem3)

  def gather(j, b):
    pltpu.async_copy(g_hbm.at[src_v.at[j]], rows.at[b], gsems[b])

  def wait_gather(j, b):
    pltpu.make_async_copy(g_hbm.at[src_v.at[j]], rows.at[b], gsems[b]).wait()

  def scatter(j, b):
    pltpu.async_copy(rows.at[b], acc_sh.at[dst_v.at[j]], ssems[b], add=True)

  def wait_scatter(j, b):
    pltpu.make_async_copy(rows.at[b], acc_sh.at[dst_v.at[j]],
                          ssems[b]).wait()

  for sec in range(SEC):
    pltpu.sync_copy(src_hbm.at[w].at[pl.ds(sec * SECC, SECC)], src_v)
    pltpu.sync_copy(dst_hbm.at[w].at[pl.ds(sec * SECC, SECC)], dst_v)

    gather(0, 0)

    def pair(p, carry):
      for b in range(2):
        j = p * 2 + b
        nb = 1 - b

        @pl.when(j + 1 < SECC)
        def _prefetch():
          @pl.when(j >= 1)
          def _():
            wait_scatter(j - 1, nb)
          gather(j + 1, nb)

        wait_gather(j, b)
        scatter(j, b)
      return carry

    lax.fori_loop(0, SECC // 2, pair, 0)

    wait_scatter(SECC - 2, 0)
    wait_scatter(SECC - 1, 1)

  plsc.subcore_barrier()
  pltpu.sync_copy(acc_sh.at[pl.ds(base, ROWS_PER_TILE)],
                  acc_out.at[c].at[pl.ds(base, ROWS_PER_TILE)])


def _scatter_rows(g, src4, dst4):
  k = pl.kernel(
      _agg_body,
      out_type=jax.ShapeDtypeStruct((NC, N_PAD, D), jnp.float32),
      mesh=_mesh(),
      scratch_types=[
          pltpu.VMEM((SECC, CHUNK), jnp.int32),
          pltpu.VMEM((SECC, CHUNK), jnp.int32),
          pltpu.VMEM((2, CHUNK, D), jnp.float32),
          pltpu.VMEM_SHARED((N_PAD, D), jnp.float32),
          pltpu.SemaphoreType.DMA,
          pltpu.SemaphoreType.DMA,
          pltpu.SemaphoreType.DMA,
          pltpu.SemaphoreType.DMA,
      ],
  )
  return k(g, src4, dst4)


BLK = 1024
GRID = N_PAD // BLK


def _mma_body(x_ref, w_ref, h_ref):
  h_ref[...] = jnp.dot(x_ref[...], w_ref[...],
                       preferred_element_type=jnp.float32)


def _mmb_body(h_ref, dp_ref, g_ref, dinv_ref):
  deg = dp_ref[0] + dp_ref[1] + 1.0
  dinv = lax.rsqrt(deg)[:, None]
  g_ref[...] = h_ref[...] * dinv
  dinv_ref[...] = dinv


def _mid_body(acc_ref, h_ref, dinv_ref, b_ref, w_ref, h2_ref, g2_ref):
  a = acc_ref[0] + acc_ref[1]
  dinv = dinv_ref[...]
  z = jnp.maximum(a * dinv + h_ref[...] * (dinv * dinv) + b_ref[...], 0.0)
  h2 = jnp.dot(z, w_ref[...], preferred_element_type=jnp.float32)
  h2_ref[...] = h2
  g2_ref[...] = h2 * dinv


def _fin_body(acc_ref, h_ref, dinv_ref, b_ref, o_ref):
  a = acc_ref[0] + acc_ref[1]
  dinv = dinv_ref[...]
  o_ref[...] = a * dinv + h_ref[...] * (dinv * dinv) + b_ref[...]


def _mma(x, W1):
  return pl.pallas_call(
      _mma_body,
      grid=(GRID,),
      in_specs=[
          pl.BlockSpec((BLK, D), lambda i: (i, 0)),
          pl.BlockSpec((D, D), lambda i: (0, 0)),
      ],
      out_specs=pl.BlockSpec((BLK, D), lambda i: (i, 0)),
      out_shape=jax.ShapeDtypeStruct((N_PAD, D), jnp.float32),
  )(x, W1)


def _mmb(h1, dp):
  return pl.pallas_call(
      _mmb_body,
      grid=(GRID,),
      in_specs=[
          pl.BlockSpec((BLK, D), lambda i: (i, 0)),
          pl.BlockSpec((NC, BLK), lambda i: (0, i)),
      ],
      out_specs=[
          pl.BlockSpec((BLK, D), lambda i: (i, 0)),
          pl.BlockSpec((BLK, 1), lambda i: (i, 0)),
      ],
      out_shape=[
          jax.ShapeDtypeStruct((N_PAD, D), jnp.float32),
          jax.ShapeDtypeStruct((N_PAD, 1), jnp.float32),
      ],
  )(h1, dp)


def _mid(acc, h1, dinv_col, b1, W2):
  return pl.pallas_call(
      _mid_body,
      grid=(GRID,),
      in_specs=[
          pl.BlockSpec((NC, BLK, D), lambda i: (0, i, 0)),
          pl.BlockSpec((BLK, D), lambda i: (i, 0)),
          pl.BlockSpec((BLK, 1), lambda i: (i, 0)),
          pl.BlockSpec((1, D), lambda i: (0, 0)),
          pl.BlockSpec((D, D), lambda i: (0, 0)),
      ],
      out_specs=[
          pl.BlockSpec((BLK, D), lambda i: (i, 0)),
          pl.BlockSpec((BLK, D), lambda i: (i, 0)),
      ],
      out_shape=[
          jax.ShapeDtypeStruct((N_PAD, D), jnp.float32),
          jax.ShapeDtypeStruct((N_PAD, D), jnp.float32),
      ],
  )(acc, h1, dinv_col, b1.reshape(1, D), W2)


def _fin(acc, h2, dinv_col, b2):
  return pl.pallas_call(
      _fin_body,
      grid=(GRID,),
      in_specs=[
          pl.BlockSpec((NC, BLK, D), lambda i: (0, i, 0)),
          pl.BlockSpec((BLK, D), lambda i: (i, 0)),
          pl.BlockSpec((BLK, 1), lambda i: (i, 0)),
          pl.BlockSpec((1, D), lambda i: (0, 0)),
      ],
      out_specs=pl.BlockSpec((BLK, D), lambda i: (i, 0)),
      out_shape=jax.ShapeDtypeStruct((N_PAD, D), jnp.float32),
  )(acc, h2, dinv_col, b2.reshape(1, D))


@jax.jit
def kernel(x, edge_index, W1, b1, W2, b2):
  src = edge_index[0]
  dst = edge_index[1]
  spare = N_PAD - N
  pad3 = E_PAD - E
  fill3 = N + (jnp.arange(pad3, dtype=jnp.int32) % spare)
  dst3 = jnp.concatenate([dst, fill3]).reshape(NW, C, CHUNK)
  pad4 = E_PAD4 - E
  fill4 = N + (jnp.arange(pad4, dtype=jnp.int32) % spare)
  src4 = jnp.concatenate([src, fill4]).reshape(NW, C4, CHUNK)
  dst4 = jnp.concatenate([dst, fill4]).reshape(NW, C4, CHUNK)

  dp = _deg_partials(dst3)
  h1 = _mma(x, W1)
  g1, dinv_col = _mmb(h1, dp)
  acc1 = _scatter_rows(g1, src4, dst4)
  h2, g2 = _mid(acc1, h1, dinv_col, b1, W2)
  acc2 = _scatter_rows(g2, src4, dst4)
  out = _fin(acc2, h2, dinv_col, b2)
  return out[:N]


# speedup vs baseline: 3.7451x; 1.0125x over previous
import jax
import jax.numpy as jnp
import numpy as np
from jax import lax
from jax.experimental import pallas as pl
from jax.experimental.pallas import tpu as pltpu
from jax.experimental.pallas import tpu_sc as plsc

NC = 2
NS = 16
NW = NC * NS
LANES = 16

N = 10000
D = 128
E = 320000
N_PAD = 10240
CHUNK = 128
ROWS_PER_TILE = N_PAD // NS
DEG_PER_TILE = N_PAD // NS
SECC = 40
SEC = 2
C4 = SEC * SECC
E_PAD4 = NW * C4 * CHUNK

_mesh = lambda: plsc.VectorSubcoreMesh(core_axis_name="c", subcore_axis_name="s")


def _deg_body(dst_hbm, out_hbm, dst_v, ones_v, zero_v, deg_sh):
  c = lax.axis_index("c")
  s = lax.axis_index("s")
  w = c * NS + s
  pltpu.sync_copy(dst_hbm.at[w], dst_v)

  def fill(i, carry):
    ones_v[pl.ds(i * LANES, LANES)] = jnp.ones((LANES,), jnp.float32)
    return carry
  lax.fori_loop(0, CHUNK // LANES, fill, 0, unroll=8)

  def zfill(i, carry):
    zero_v[pl.ds(i * LANES, LANES)] = jnp.zeros((LANES,), jnp.float32)
    return carry
  lax.fori_loop(0, DEG_PER_TILE // LANES, zfill, 0, unroll=8)

  pltpu.sync_copy(zero_v, deg_sh.at[pl.ds(s * DEG_PER_TILE, DEG_PER_TILE)])
  plsc.subcore_barrier()

  def count(j, carry):
    pltpu.sync_copy(ones_v, deg_sh.at[dst_v.at[j]], add=True)
    return carry
  lax.fori_loop(0, C4, count, 0)

  plsc.subcore_barrier()
  pltpu.sync_copy(deg_sh.at[pl.ds(s * DEG_PER_TILE, DEG_PER_TILE)],
                  out_hbm.at[c].at[pl.ds(s * DEG_PER_TILE, DEG_PER_TILE)])


def _deg_partials(dst3):
  k = pl.kernel(
      _deg_body,
      out_type=jax.ShapeDtypeStruct((NC, N_PAD), jnp.float32),
      mesh=_mesh(),
      scratch_types=[
          pltpu.VMEM((C4, CHUNK), jnp.int32),
          pltpu.VMEM((CHUNK,), jnp.float32),
          pltpu.VMEM((DEG_PER_TILE,), jnp.float32),
          pltpu.VMEM_SHARED((N_PAD,), jnp.float32),
      ],
  )
  return k(dst3)


def _agg_body(g_hbm, src_hbm, dst_hbm, acc_out, src_v, dst_v, rows, acc_sh,
              sem0, sem1, sem2, sem3):
  c = lax.axis_index("c")
  s = lax.axis_index("s")
  w = c * NS + s

  def zero(i, carry):
    for k in range(D // LANES):
      rows[0, i, pl.ds(k * LANES, LANES)] = jnp.zeros((LANES,), jnp.float32)
    return carry
  lax.fori_loop(0, CHUNK, zero, 0, unroll=4)
  base = s * ROWS_PER_TILE
  for t in range(ROWS_PER_TILE // CHUNK):
    pltpu.sync_copy(rows.at[0], acc_sh.at[pl.ds(base + t * CHUNK, CHUNK)])
  plsc.subcore_barrier()

  gsems = (sem0, sem1)
  ssems = (sem2, sem3)

  def gather(j, b):
    pltpu.async_copy(g_hbm.at[src_v.at[j]], rows.at[b], gsems[b])

  def wait_gather(j, b):
    pltpu.make_async_copy(g_hbm.at[src_v.at[j]], rows.at[b], gsems[b]).wait()

  def scatter(j, b):
    pltpu.async_copy(rows.at[b], acc_sh.at[dst_v.at[j]], ssems[b], add=True)

  def wait_scatter(j, b):
    pltpu.make_async_copy(rows.at[b], acc_sh.at[dst_v.at[j]],
                          ssems[b]).wait()

  for sec in range(SEC):
    pltpu.sync_copy(src_hbm.at[w].at[pl.ds(sec * SECC, SECC)], src_v)
    pltpu.sync_copy(dst_hbm.at[w].at[pl.ds(sec * SECC, SECC)], dst_v)

    gather(0, 0)

    def pair(p, carry):
      for b in range(2):
        j = p * 2 + b
        nb = 1 - b

        @pl.when(j + 1 < SECC)
        def _prefetch():
          @pl.when(j >= 1)
          def _():
            wait_scatter(j - 1, nb)
          gather(j + 1, nb)

        wait_gather(j, b)
        scatter(j, b)
      return carry

    lax.fori_loop(0, SECC // 2, pair, 0)

    wait_scatter(SECC - 2, 0)
    wait_scatter(SECC - 1, 1)

  plsc.subcore_barrier()
  pltpu.sync_copy(acc_sh.at[pl.ds(base, ROWS_PER_TILE)],
                  acc_out.at[c].at[pl.ds(base, ROWS_PER_TILE)])


def _scatter_rows(g, src4, dst4):
  k = pl.kernel(
      _agg_body,
      out_type=jax.ShapeDtypeStruct((NC, N_PAD, D), jnp.float32),
      mesh=_mesh(),
      scratch_types=[
          pltpu.VMEM((SECC, CHUNK), jnp.int32),
          pltpu.VMEM((SECC, CHUNK), jnp.int32),
          pltpu.VMEM((2, CHUNK, D), jnp.float32),
          pltpu.VMEM_SHARED((N_PAD, D), jnp.float32),
          pltpu.SemaphoreType.DMA,
          pltpu.SemaphoreType.DMA,
          pltpu.SemaphoreType.DMA,
          pltpu.SemaphoreType.DMA,
      ],
  )
  return k(g, src4, dst4)


BLK = 1024
GRID = N_PAD // BLK


def _mma_body(x_ref, w_ref, h_ref):
  h_ref[...] = jnp.dot(x_ref[...], w_ref[...],
                       preferred_element_type=jnp.float32)


def _mmb_body(h_ref, dp_ref, g_ref, dinv_ref):
  deg = dp_ref[0] + dp_ref[1] + 1.0
  dinv = lax.rsqrt(deg)[:, None]
  g_ref[...] = h_ref[...] * dinv
  dinv_ref[...] = dinv


def _mid_body(acc_ref, h_ref, dinv_ref, b_ref, w_ref, h2_ref, g2_ref):
  a = acc_ref[0] + acc_ref[1]
  dinv = dinv_ref[...]
  z = jnp.maximum(a * dinv + h_ref[...] * (dinv * dinv) + b_ref[...], 0.0)
  h2 = jnp.dot(z, w_ref[...], preferred_element_type=jnp.float32)
  h2_ref[...] = h2
  g2_ref[...] = h2 * dinv


def _fin_body(acc_ref, h_ref, dinv_ref, b_ref, o_ref):
  a = acc_ref[0] + acc_ref[1]
  dinv = dinv_ref[...]
  o_ref[...] = a * dinv + h_ref[...] * (dinv * dinv) + b_ref[...]


def _mma(x, W1):
  return pl.pallas_call(
      _mma_body,
      grid=(GRID,),
      in_specs=[
          pl.BlockSpec((BLK, D), lambda i: (i, 0)),
          pl.BlockSpec((D, D), lambda i: (0, 0)),
      ],
      out_specs=pl.BlockSpec((BLK, D), lambda i: (i, 0)),
      out_shape=jax.ShapeDtypeStruct((N_PAD, D), jnp.float32),
  )(x, W1)


def _mmb(h1, dp):
  return pl.pallas_call(
      _mmb_body,
      grid=(GRID,),
      in_specs=[
          pl.BlockSpec((BLK, D), lambda i: (i, 0)),
          pl.BlockSpec((NC, BLK), lambda i: (0, i)),
      ],
      out_specs=[
          pl.BlockSpec((BLK, D), lambda i: (i, 0)),
          pl.BlockSpec((BLK, 1), lambda i: (i, 0)),
      ],
      out_shape=[
          jax.ShapeDtypeStruct((N_PAD, D), jnp.float32),
          jax.ShapeDtypeStruct((N_PAD, 1), jnp.float32),
      ],
  )(h1, dp)


def _mid(acc, h1, dinv_col, b1, W2):
  return pl.pallas_call(
      _mid_body,
      grid=(GRID,),
      in_specs=[
          pl.BlockSpec((NC, BLK, D), lambda i: (0, i, 0)),
          pl.BlockSpec((BLK, D), lambda i: (i, 0)),
          pl.BlockSpec((BLK, 1), lambda i: (i, 0)),
          pl.BlockSpec((1, D), lambda i: (0, 0)),
          pl.BlockSpec((D, D), lambda i: (0, 0)),
      ],
      out_specs=[
          pl.BlockSpec((BLK, D), lambda i: (i, 0)),
          pl.BlockSpec((BLK, D), lambda i: (i, 0)),
      ],
      out_shape=[
          jax.ShapeDtypeStruct((N_PAD, D), jnp.float32),
          jax.ShapeDtypeStruct((N_PAD, D), jnp.float32),
      ],
  )(acc, h1, dinv_col, b1.reshape(1, D), W2)


def _fin(acc, h2, dinv_col, b2):
  return pl.pallas_call(
      _fin_body,
      grid=(GRID,),
      in_specs=[
          pl.BlockSpec((NC, BLK, D), lambda i: (0, i, 0)),
          pl.BlockSpec((BLK, D), lambda i: (i, 0)),
          pl.BlockSpec((BLK, 1), lambda i: (i, 0)),
          pl.BlockSpec((1, D), lambda i: (0, 0)),
      ],
      out_specs=pl.BlockSpec((BLK, D), lambda i: (i, 0)),
      out_shape=jax.ShapeDtypeStruct((N, D), jnp.float32),
  )(acc, h2, dinv_col, b2.reshape(1, D))


@jax.jit
def kernel(x, edge_index, W1, b1, W2, b2):
  src = edge_index[0]
  dst = edge_index[1]
  spare = N_PAD - N
  pad4 = E_PAD4 - E
  fill4 = jnp.asarray(np.int32(N) + np.arange(pad4, dtype=np.int32) % spare)
  src4 = jnp.concatenate([src, fill4]).reshape(NW, C4, CHUNK)
  dst4 = jnp.concatenate([dst, fill4]).reshape(NW, C4, CHUNK)

  dp = _deg_partials(dst4)
  h1 = _mma(x, W1)
  g1, dinv_col = _mmb(h1, dp)
  acc1 = _scatter_rows(g1, src4, dst4)
  h2, g2 = _mid(acc1, h1, dinv_col, b1, W2)
  acc2 = _scatter_rows(g2, src4, dst4)
  return _fin(acc2, h2, dinv_col, b2)


# speedup vs baseline: 3.8259x; 1.0216x over previous
import jax
import jax.numpy as jnp
import numpy as np
from jax import lax
from jax.experimental import pallas as pl
from jax.experimental.pallas import tpu as pltpu
from jax.experimental.pallas import tpu_sc as plsc

NC = 2
NS = 16
NW = NC * NS
LANES = 16

N = 10000
D = 128
E = 320000
N_PAD = 10240
CHUNK = 128
ROWS_PER_TILE = N_PAD // NS
DEG_PER_TILE = N_PAD // NS
SECC = 40
SEC = 2
C4 = SEC * SECC
E_PAD4 = NW * C4 * CHUNK
RC = E // CHUNK
TAIL_REAL = RC - (NW - 1) * C4
PADC = NW * C4 - RC

_mesh = lambda: plsc.VectorSubcoreMesh(core_axis_name="c", subcore_axis_name="s")


def _deg_body(dst_hbm, pad_hbm, out_hbm, dst_v, ones_v, zero_v, deg_sh):
  c = lax.axis_index("c")
  s = lax.axis_index("s")
  w = c * NS + s

  @pl.when(w < NW - 1)
  def _():
    pltpu.sync_copy(dst_hbm.at[pl.ds(w * C4, C4)], dst_v)

  @pl.when(w == NW - 1)
  def _():
    pltpu.sync_copy(pad_hbm.at[1], dst_v)

  def fill(i, carry):
    ones_v[pl.ds(i * LANES, LANES)] = jnp.ones((LANES,), jnp.float32)
    return carry
  lax.fori_loop(0, CHUNK // LANES, fill, 0, unroll=8)

  def zfill(i, carry):
    zero_v[pl.ds(i * LANES, LANES)] = jnp.zeros((LANES,), jnp.float32)
    return carry
  lax.fori_loop(0, DEG_PER_TILE // LANES, zfill, 0, unroll=8)

  pltpu.sync_copy(zero_v, deg_sh.at[pl.ds(s * DEG_PER_TILE, DEG_PER_TILE)])
  plsc.subcore_barrier()

  def count(j, carry):
    pltpu.sync_copy(ones_v, deg_sh.at[dst_v.at[j]], add=True)
    return carry
  lax.fori_loop(0, C4, count, 0)

  plsc.subcore_barrier()
  pltpu.sync_copy(deg_sh.at[pl.ds(s * DEG_PER_TILE, DEG_PER_TILE)],
                  out_hbm.at[c].at[pl.ds(s * DEG_PER_TILE, DEG_PER_TILE)])


def _deg_partials(dstE, padc):
  k = pl.kernel(
      _deg_body,
      out_type=jax.ShapeDtypeStruct((NC, N_PAD), jnp.float32),
      mesh=_mesh(),
      scratch_types=[
          pltpu.VMEM((C4, CHUNK), jnp.int32),
          pltpu.VMEM((CHUNK,), jnp.float32),
          pltpu.VMEM((DEG_PER_TILE,), jnp.float32),
          pltpu.VMEM_SHARED((N_PAD,), jnp.float32),
      ],
  )
  return k(dstE, padc)


def _agg_body(g_hbm, src_hbm, dst_hbm, pad_hbm, acc_out, src_v, dst_v, rows,
              acc_sh, sem0, sem1, sem2, sem3):
  c = lax.axis_index("c")
  s = lax.axis_index("s")
  w = c * NS + s

  def zero(i, carry):
    for k in range(D // LANES):
      rows[0, i, pl.ds(k * LANES, LANES)] = jnp.zeros((LANES,), jnp.float32)
    return carry
  lax.fori_loop(0, CHUNK, zero, 0, unroll=4)
  base = s * ROWS_PER_TILE
  for t in range(ROWS_PER_TILE // CHUNK):
    pltpu.sync_copy(rows.at[0], acc_sh.at[pl.ds(base + t * CHUNK, CHUNK)])
  plsc.subcore_barrier()

  gsems = (sem0, sem1)
  ssems = (sem2, sem3)

  def gather(j, b):
    pltpu.async_copy(g_hbm.at[src_v.at[j]], rows.at[b], gsems[b])

  def wait_gather(j, b):
    pltpu.make_async_copy(g_hbm.at[src_v.at[j]], rows.at[b], gsems[b]).wait()

  def scatter(j, b):
    pltpu.async_copy(rows.at[b], acc_sh.at[dst_v.at[j]], ssems[b], add=True)

  def wait_scatter(j, b):
    pltpu.make_async_copy(rows.at[b], acc_sh.at[dst_v.at[j]],
                          ssems[b]).wait()

  for sec in range(SEC):
    start = w * C4 + sec * SECC

    @pl.when(w < NW - 1)
    def _():
      pltpu.sync_copy(src_hbm.at[pl.ds(start, SECC)], src_v)
      pltpu.sync_copy(dst_hbm.at[pl.ds(start, SECC)], dst_v)

    @pl.when(w == NW - 1)
    def _():
      pltpu.sync_copy(pad_hbm.at[0].at[pl.ds(sec * SECC, SECC)], src_v)
      pltpu.sync_copy(pad_hbm.at[1].at[pl.ds(sec * SECC, SECC)], dst_v)

    gather(0, 0)

    def pair(p, carry):
      for b in range(2):
        j = p * 2 + b
        nb = 1 - b

        @pl.when(j + 1 < SECC)
        def _prefetch():
          @pl.when(j >= 1)
          def _():
            wait_scatter(j - 1, nb)
          gather(j + 1, nb)

        wait_gather(j, b)
        scatter(j, b)
      return carry

    lax.fori_loop(0, SECC // 2, pair, 0)

    wait_scatter(SECC - 2, 0)
    wait_scatter(SECC - 1, 1)

  plsc.subcore_barrier()
  pltpu.sync_copy(acc_sh.at[pl.ds(base, ROWS_PER_TILE)],
                  acc_out.at[c].at[pl.ds(base, ROWS_PER_TILE)])


def _scatter_rows(g, srcE, dstE, padc):
  k = pl.kernel(
      _agg_body,
      out_type=jax.ShapeDtypeStruct((NC, N_PAD, D), jnp.float32),
      mesh=_mesh(),
      scratch_types=[
          pltpu.VMEM((SECC, CHUNK), jnp.int32),
          pltpu.VMEM((SECC, CHUNK), jnp.int32),
          pltpu.VMEM((2, CHUNK, D), jnp.float32),
          pltpu.VMEM_SHARED((N_PAD, D), jnp.float32),
          pltpu.SemaphoreType.DMA,
          pltpu.SemaphoreType.DMA,
          pltpu.SemaphoreType.DMA,
          pltpu.SemaphoreType.DMA,
      ],
  )
  return k(g, srcE, dstE, padc)


BLK = 2048
GRID = N_PAD // BLK


def _mma_body(x_ref, w_ref, h_ref):
  h_ref[...] = jnp.dot(x_ref[...], w_ref[...],
                       preferred_element_type=jnp.float32)


def _mmb_body(h_ref, dp_ref, g_ref, dinv_ref):
  deg = dp_ref[0] + dp_ref[1] + 1.0
  dinv = lax.rsqrt(deg)[:, None]
  g_ref[...] = h_ref[...] * dinv
  dinv_ref[...] = dinv


def _mid_body(acc_ref, h_ref, dinv_ref, b_ref, w_ref, h2_ref, g2_ref):
  a = acc_ref[0] + acc_ref[1]
  dinv = dinv_ref[...]
  z = jnp.maximum(a * dinv + h_ref[...] * (dinv * dinv) + b_ref[...], 0.0)
  h2 = jnp.dot(z, w_ref[...], preferred_element_type=jnp.float32)
  h2_ref[...] = h2
  g2_ref[...] = h2 * dinv


def _fin_body(acc_ref, h_ref, dinv_ref, b_ref, o_ref):
  a = acc_ref[0] + acc_ref[1]
  dinv = dinv_ref[...]
  o_ref[...] = a * dinv + h_ref[...] * (dinv * dinv) + b_ref[...]


def _mma(x, W1):
  return pl.pallas_call(
      _mma_body,
      grid=(GRID,),
      in_specs=[
          pl.BlockSpec((BLK, D), lambda i: (i, 0)),
          pl.BlockSpec((D, D), lambda i: (0, 0)),
      ],
      out_specs=pl.BlockSpec((BLK, D), lambda i: (i, 0)),
      out_shape=jax.ShapeDtypeStruct((N_PAD, D), jnp.float32),
  )(x, W1)


def _mmb(h1, dp):
  return pl.pallas_call(
      _mmb_body,
      grid=(GRID,),
      in_specs=[
          pl.BlockSpec((BLK, D), lambda i: (i, 0)),
          pl.BlockSpec((NC, BLK), lambda i: (0, i)),
      ],
      out_specs=[
          pl.BlockSpec((BLK, D), lambda i: (i, 0)),
          pl.BlockSpec((BLK, 1), lambda i: (i, 0)),
      ],
      out_shape=[
          jax.ShapeDtypeStruct((N_PAD, D), jnp.float32),
          jax.ShapeDtypeStruct((N_PAD, 1), jnp.float32),
      ],
  )(h1, dp)


def _mid(acc, h1, dinv_col, b1, W2):
  return pl.pallas_call(
      _mid_body,
      grid=(GRID,),
      in_specs=[
          pl.BlockSpec((NC, BLK, D), lambda i: (0, i, 0)),
          pl.BlockSpec((BLK, D), lambda i: (i, 0)),
          pl.BlockSpec((BLK, 1), lambda i: (i, 0)),
          pl.BlockSpec((1, D), lambda i: (0, 0)),
          pl.BlockSpec((D, D), lambda i: (0, 0)),
      ],
      out_specs=[
          pl.BlockSpec((BLK, D), lambda i: (i, 0)),
          pl.BlockSpec((BLK, D), lambda i: (i, 0)),
      ],
      out_shape=[
          jax.ShapeDtypeStruct((N_PAD, D), jnp.float32),
          jax.ShapeDtypeStruct((N_PAD, D), jnp.float32),
      ],
  )(acc, h1, dinv_col, b1.reshape(1, D), W2)


def _fin(acc, h2, dinv_col, b2):
  return pl.pallas_call(
      _fin_body,
      grid=(GRID,),
      in_specs=[
          pl.BlockSpec((NC, BLK, D), lambda i: (0, i, 0)),
          pl.BlockSpec((BLK, D), lambda i: (i, 0)),
          pl.BlockSpec((BLK, 1), lambda i: (i, 0)),
          pl.BlockSpec((1, D), lambda i: (0, 0)),
      ],
      out_specs=pl.BlockSpec((BLK, D), lambda i: (i, 0)),
      out_shape=jax.ShapeDtypeStruct((N, D), jnp.float32),
  )(acc, h2, dinv_col, b2.reshape(1, D))


@jax.jit
def kernel(x, edge_index, W1, b1, W2, b2):
  srcE = edge_index[0].reshape(RC, CHUNK)
  dstE = edge_index[1].reshape(RC, CHUNK)
  spare = N_PAD - N
  padc = jnp.asarray(
      np.int32(N) + np.arange(PADC * CHUNK, dtype=np.int32) % spare
  ).reshape(PADC, CHUNK)
  tail = jnp.stack([
      jnp.concatenate([srcE[RC - TAIL_REAL:], padc]),
      jnp.concatenate([dstE[RC - TAIL_REAL:], padc]),
  ])

  dp = _deg_partials(dstE, tail)
  h1 = _mma(x, W1)
  g1, dinv_col = _mmb(h1, dp)
  acc1 = _scatter_rows(g1, srcE, dstE, tail)
  h2, g2 = _mid(acc1, h1, dinv_col, b1, W2)
  acc2 = _scatter_rows(g2, srcE, dstE, tail)
  return _fin(acc2, h2, dinv_col, b2)
